# Initial kernel scaffold; baseline (speedup 1.0000x reference)
import jax
import jax.numpy as jnp
from jax.experimental import pallas as pl


def kernel(x, idx):
    raise NotImplementedError("write your pallas kernel here")


# speedup vs baseline: 1.9784x; 1.9784x over previous
import functools

import jax
import jax.numpy as jnp
from jax import lax
from jax.experimental import pallas as pl
from jax.experimental.pallas import tpu as pltpu
from jax.experimental.pallas import tpu_sc as plsc

N_IDX = 100000
D = 128
C = 128
NW = 32
N_CHUNKS = -(-N_IDX // C)
B_PAD = N_CHUNKS * C
MAX_CHUNKS_PER_W = -(-N_CHUNKS // NW)

_mesh = plsc.VectorSubcoreMesh(core_axis_name="c", subcore_axis_name="s")


@functools.partial(
    pl.kernel,
    mesh=_mesh,
    out_type=jax.ShapeDtypeStruct((B_PAD, D), jnp.float32),
    scratch_types=[
        pltpu.VMEM((C,), jnp.int32),
        pltpu.VMEM((C, D), jnp.float32),
        pltpu.SemaphoreType.DMA,
    ],
)
def _sc_gather(x_hbm, idx_hbm, out_hbm, idx_v, rows_v, sem):
    w = lax.axis_index("s") * 2 + lax.axis_index("c")

    def body(j, carry):
        g = j * NW + w

        @pl.when(g < N_CHUNKS)
        def _():
            pltpu.sync_copy(idx_hbm.at[g], idx_v)
            pltpu.async_copy(x_hbm.at[idx_v], rows_v, sem).wait()
            pltpu.sync_copy(rows_v, out_hbm.at[pl.ds(g * C, C)])

        return carry

    lax.fori_loop(0, MAX_CHUNKS_PER_W, body, 0)


def kernel(x, idx):
    idx32 = idx.astype(jnp.int32)
    idx_pad = jnp.zeros((B_PAD,), jnp.int32).at[:N_IDX].set(idx32)
    out = _sc_gather(x, idx_pad.reshape(N_CHUNKS, C))
    return out[:N_IDX]
